# baseline (speedup 1.0000x reference)
import functools

import jax
import jax.numpy as jnp
from jax import lax
from jax.experimental import pallas as pl
from jax.experimental.pallas import tpu as pltpu
from jax.experimental.pallas import tpu_sc as plsc

_N_ITEMS = 50000
_N_USERS = 10000
_EMB = 100
_D = 128
_NI_PAD = 51200
_NU_PAD = 10240
_NC = 2
_NS = 16
_L = 16
_NW = _NC * _NS
_BLK = 2000
_CAP = 128
_STR = _CAP + _L

_NNZ_A = 800000
_NNZ_A_PAD = 832000
_NNZ_U = 320000

_CP = pltpu.CompilerParams(needs_layout_passes=False)


def _make_bin(nnz_pad, nch, chunk, cap_b):
    span = nnz_pad // _NW
    nblk = span // _BLK
    ngrp = _BLK // _L
    assert span % _BLK == 0 and nch <= 16
    mesh = plsc.VectorSubcoreMesh(core_axis_name="c", subcore_axis_name="s")

    @functools.partial(
        pl.kernel,
        out_type=(jax.ShapeDtypeStruct((_NW, nch, cap_b, 8, _CAP), jnp.int32),
                  jax.ShapeDtypeStruct((_NW, 16), jnp.int32)),
        mesh=mesh,
        compiler_params=_CP,
        scratch_types=[
            pltpu.VMEM((_BLK,), jnp.int32),
            pltpu.VMEM((_BLK,), jnp.int32),
            pltpu.VMEM((_BLK,), jnp.float32),
            pltpu.VMEM((nch * _STR,), jnp.int32),
            pltpu.VMEM((nch * _STR,), jnp.int32),
            pltpu.VMEM((nch * _STR,), jnp.float32),
            pltpu.VMEM((8, _CAP), jnp.int32),
            pltpu.VMEM((16,), jnp.int32),
            pltpu.SMEM((48,), jnp.int32),
        ],
    )
    def binner(row_h, col_h, val_h, bins_h, counts_h,
               rowb, colb, valb, colf, ridxf, valf, tb, cbv, st):
        cid = lax.axis_index("c")
        sid = lax.axis_index("s")
        wid = 2 * sid + cid
        z16i = jnp.zeros((_L,), jnp.int32)
        z16f = jnp.zeros((_L,), jnp.float32)
        iota = lax.iota(jnp.int32, _L)

        for i in range(nch * _STR // _L):
            colf[pl.ds(i * _L, _L)] = z16i
            ridxf[pl.ds(i * _L, _L)] = z16i
            valf[pl.ds(i * _L, _L)] = z16f
        for k in range(nch):
            st[k] = 0
            st[16 + k] = 0

        def flush(k):
            for i in range(_CAP // _L):
                tb[0, pl.ds(i * _L, _L)] = colf[pl.ds(k * _STR + i * _L, _L)]
                tb[1, pl.ds(i * _L, _L)] = ridxf[pl.ds(k * _STR + i * _L, _L)]
                tb[2, pl.ds(i * _L, _L)] = plsc.bitcast(
                    valf[pl.ds(k * _STR + i * _L, _L)], jnp.int32)
            nb = st[16 + k]
            pltpu.sync_copy(tb, bins_h.at[wid, k, nb])
            st[16 + k] = nb + 1
            for i in range(_CAP // _L):
                valf[pl.ds(k * _STR + i * _L, _L)] = z16f
            st[k] = 0

        off0 = wid * span

        def _grp(g, c):
            rv = rowb[pl.ds(g * _L, _L)]
            cv = colb[pl.ds(g * _L, _L)]
            vv = valb[pl.ds(g * _L, _L)]
            for k in range(nch):
                base = k * chunk
                m = (rv >= base) & (rv < base + chunk)

                @pl.when(st[k] > _CAP - _L)
                def _():
                    flush(k)

                cnt = st[k]
                cs = lax.cumsum(m.astype(jnp.int32))
                pos = jnp.where(m, k * _STR + cnt + cs - 1, k * _STR + _CAP)
                plsc.store_scatter(colf, [pos], cv)
                plsc.store_scatter(ridxf, [pos], rv - base)
                plsc.store_scatter(valf, [pos], vv)
                st[k] = cnt + cs[_L - 1]
            return c

        def _blk(b, c):
            off = off0 + b * _BLK
            pltpu.sync_copy(row_h.at[pl.ds(off, _BLK)], rowb)
            pltpu.sync_copy(col_h.at[pl.ds(off, _BLK)], colb)
            pltpu.sync_copy(val_h.at[pl.ds(off, _BLK)], valb)
            lax.fori_loop(0, ngrp, _grp, 0)
            return c
        lax.fori_loop(0, nblk, _blk, 0)

        for k in range(nch):
            @pl.when(st[k] > 0)
            def _():
                flush(k)

        for i in range(_CAP // _L):
            tb[0, pl.ds(i * _L, _L)] = z16i
            tb[1, pl.ds(i * _L, _L)] = z16i
            tb[2, pl.ds(i * _L, _L)] = z16i
        for k in range(nch):
            for _extra in range(3):
                @pl.when(lax.rem(st[16 + k], 4) != 0)
                def _():
                    pltpu.sync_copy(tb, bins_h.at[wid, k, st[16 + k]])
                    st[16 + k] = st[16 + k] + 1

        cvec = z16i
        for k in range(nch):
            cvec = jnp.where(iota == k, st[16 + k], cvec)
        cbv[pl.ds(0, _L)] = cvec
        pltpu.sync_copy(cbv, counts_h.at[wid])

    return binner


def _make_layer(nch, chunk, cap_b, n_out_pad):
    cps = nch // _NC
    rows_per_tile = chunk // _NS
    zfull, zrem = divmod(rows_per_tile, _CAP)
    mesh = plsc.VectorSubcoreMesh(core_axis_name="c", subcore_axis_name="s")

    @functools.partial(
        pl.kernel,
        out_type=jax.ShapeDtypeStruct((n_out_pad, _D), jnp.float32),
        mesh=mesh,
        compiler_params=_CP,
        scratch_types=[
            pltpu.VMEM_SHARED((chunk, _D), jnp.float32),
            pltpu.VMEM((2, 4, 8, _CAP), jnp.int32),
            pltpu.VMEM((4 * _CAP, _D), jnp.float32),
            pltpu.VMEM((16,), jnp.int32),
            pltpu.SemaphoreType.DMA,
            pltpu.SemaphoreType.DMA,
            pltpu.SemaphoreType.DMA,
        ],
    )
    def layer(bins_h, counts_h, x_h, out_h, accum, tbs, G, cbv,
              sem_t, sem_g, sem_s):
        cid = lax.axis_index("c")
        sid = lax.axis_index("s")
        z16f = jnp.zeros((_L,), jnp.float32)
        iota = lax.iota(jnp.int32, _L)
        row0 = sid * rows_per_tile

        for k_i in range(cps):
            k = cid * cps + k_i
            def _zg(i, c):
                for d in range(_D // _L):
                    G[i, pl.ds(d * _L, _L)] = z16f
                return c
            lax.fori_loop(0, _CAP, _zg, 0)
            for z in range(zfull):
                pltpu.sync_copy(G.at[pl.ds(0, _CAP)],
                                accum.at[pl.ds(row0 + z * _CAP, _CAP)])
            if zrem:
                pltpu.sync_copy(G.at[pl.ds(0, zrem)],
                                accum.at[pl.ds(row0 + zfull * _CAP, zrem)])
            plsc.subcore_barrier()

            for j in range(2):
                w = 2 * sid + j
                pltpu.sync_copy(counts_h.at[w], cbv)
                nbv = cbv[pl.ds(0, _L)]
                nb = jnp.sum(jnp.where(iota == k, nbv, 0))
                ng = nb // 4

                @pl.when(ng > 0)
                def _():
                    pltpu.sync_copy(bins_h.at[w, k, pl.ds(0, 4)], tbs.at[0])

                def _grp4(g, c):
                    buf = lax.rem(g, 2)

                    @pl.when(g > 0)
                    def _():
                        pltpu.make_async_copy(
                            bins_h.at[w, k, pl.ds(0, 4)],
                            tbs.at[buf], sem_t).wait()

                    @pl.when(g + 1 < ng)
                    def _():
                        pltpu.async_copy(bins_h.at[w, k, pl.ds(4 * (g + 1), 4)],
                                         tbs.at[1 - buf], sem_t)

                    gd = [pltpu.async_copy(x_h.at[tbs.at[buf, q, 0]],
                                           G.at[pl.ds(q * _CAP, _CAP)], sem_g)
                          for q in range(4)]
                    for d_ in gd:
                        d_.wait()

                    for q in range(4):
                        def _sc(g2, c2, q=q):
                            vv = plsc.bitcast(
                                tbs[buf, q, 2, pl.ds(g2 * _L, _L)],
                                jnp.float32)
                            for r in range(_L):
                                jrow = q * _CAP + g2 * _L + r
                                vs = z16f + vv[r]
                                for d in range(_D // _L):
                                    G[jrow, pl.ds(d * _L, _L)] = (
                                        G[jrow, pl.ds(d * _L, _L)] * vs)
                            return c2
                        lax.fori_loop(0, _CAP // _L, _sc, 0)

                    sd = [pltpu.async_copy(G.at[pl.ds(q * _CAP, _CAP)],
                                           accum.at[tbs.at[buf, q, 1]],
                                           sem_s, add=True)
                          for q in range(4)]
                    for d_ in sd:
                        d_.wait()
                    return c
                lax.fori_loop(0, ng, _grp4, 0)
            plsc.subcore_barrier()

            base = k * chunk
            for z in range(zfull):
                pltpu.sync_copy(accum.at[pl.ds(row0 + z * _CAP, _CAP)],
                                out_h.at[pl.ds(base + row0 + z * _CAP, _CAP)])
            if zrem:
                pltpu.sync_copy(
                    accum.at[pl.ds(row0 + zfull * _CAP, zrem)],
                    out_h.at[pl.ds(base + row0 + zfull * _CAP, zrem)])

    return layer


_CAPB_A = (_NNZ_A_PAD // _NW) // 113 + 6
_CAPB_U = (_NNZ_U // _NW) // 113 + 6
_bin_adj = _make_bin(_NNZ_A_PAD, 8, 6400, _CAPB_A)
_bin_usr = _make_bin(_NNZ_U, 2, 5120, _CAPB_U)
_layer_adj = _make_layer(8, 6400, _CAPB_A, _NI_PAD)
_layer_usr = _make_layer(2, 5120, _CAPB_U, _NU_PAD)

_gmesh = plsc.VectorSubcoreMesh(core_axis_name="c", subcore_axis_name="s")


@functools.partial(
    pl.kernel,
    out_type=jax.ShapeDtypeStruct((1024, _D), jnp.float32),
    mesh=_gmesh,
    compiler_params=_CP,
    scratch_types=[
        pltpu.VMEM((32,), jnp.int32),
        pltpu.VMEM((32, _D), jnp.float32),
        pltpu.SemaphoreType.DMA,
    ],
)
def _gather_users(user_h, tab_h, out_h, idxb, g32, sem):
    wid = lax.axis_index("s") * _NC + lax.axis_index("c")
    b0 = wid * 32
    pltpu.sync_copy(user_h.at[pl.ds(b0, 32)], idxb)
    pltpu.async_copy(tab_h.at[idxb], g32, sem).wait()
    pltpu.sync_copy(g32, out_h.at[pl.ds(b0, 32)])


def _sum4(a, b, c, d):
    def body(a_r, b_r, c_r, d_r, o_r):
        o_r[...] = a_r[...] + b_r[...] + c_r[...] + d_r[...]
    n = a.shape[0]
    blkr = 512
    return pl.pallas_call(
        body,
        grid=(n // blkr,),
        in_specs=[pl.BlockSpec((blkr, _D), lambda i: (i, 0))] * 4,
        out_specs=pl.BlockSpec((blkr, _D), lambda i: (i, 0)),
        out_shape=jax.ShapeDtypeStruct((n, _D), jnp.float32),
    )(a, b, c, d)


def kernel(adj_row, adj_col, adj_val, u_row, u_col, u_val, ishist,
           hist_item, hist_len, embedding, user_embedding, user):
    adj_row = adj_row.astype(jnp.int32)
    adj_col = adj_col.astype(jnp.int32)
    u_row = u_row.astype(jnp.int32)
    u_col = u_col.astype(jnp.int32)
    user = user.astype(jnp.int32)

    npad = _NNZ_A_PAD - _NNZ_A
    adj_row_p = jnp.pad(adj_row, (0, npad), constant_values=_NI_PAD)
    adj_col_p = jnp.pad(adj_col, (0, npad))
    adj_val_p = jnp.pad(adj_val, (0, npad))

    x0 = jnp.pad(embedding, ((0, _NI_PAD - _N_ITEMS), (0, _D - _EMB)))

    bins_a, counts_a = _bin_adj(adj_row_p, adj_col_p, adj_val_p)
    c1 = _layer_adj(bins_a, counts_a, x0)
    c2 = _layer_adj(bins_a, counts_a, c1)
    c3 = _layer_adj(bins_a, counts_a, c2)
    fin = _sum4(x0, c1, c2, c3)

    bins_u, counts_u = _bin_usr(u_row, u_col, u_val)
    utab = _layer_usr(bins_u, counts_u, fin)
    ue = _gather_users(user, utab)
    return fin[:_N_ITEMS, :_EMB], ue[:, :_EMB]


# speedup vs baseline: 1.0005x; 1.0005x over previous
import functools

import jax
import jax.numpy as jnp
from jax import lax
from jax.experimental import pallas as pl
from jax.experimental.pallas import tpu as pltpu
from jax.experimental.pallas import tpu_sc as plsc

_N_ITEMS = 50000
_N_USERS = 10000
_EMB = 100
_D = 128
_NI_PAD = 51200
_NU_PAD = 10240
_NC = 2
_NS = 16
_L = 16
_NW = _NC * _NS
_BLK = 2000
_CAP = 128
_STR = _CAP + _L

_NNZ_A = 800000
_NNZ_A_PAD = 832000
_NNZ_U = 320000

_CP = pltpu.CompilerParams(needs_layout_passes=False)


def _make_bin(nnz_pad, nch, chunk, cap_b):
    span = nnz_pad // _NW
    nblk = span // _BLK
    ngrp = _BLK // _L
    assert span % _BLK == 0 and nch <= 16
    mesh = plsc.VectorSubcoreMesh(core_axis_name="c", subcore_axis_name="s")

    @functools.partial(
        pl.kernel,
        out_type=(jax.ShapeDtypeStruct((_NW, nch, cap_b, 8, _CAP), jnp.int32),
                  jax.ShapeDtypeStruct((_NW, 16), jnp.int32)),
        mesh=mesh,
        compiler_params=_CP,
        scratch_types=[
            pltpu.VMEM((_BLK,), jnp.int32),
            pltpu.VMEM((_BLK,), jnp.int32),
            pltpu.VMEM((_BLK,), jnp.float32),
            pltpu.VMEM((nch * _STR,), jnp.int32),
            pltpu.VMEM((nch * _STR,), jnp.int32),
            pltpu.VMEM((nch * _STR,), jnp.float32),
            pltpu.VMEM((8, _CAP), jnp.int32),
            pltpu.VMEM((16,), jnp.int32),
            pltpu.SMEM((48,), jnp.int32),
        ],
    )
    def binner(row_h, col_h, val_h, bins_h, counts_h,
               rowb, colb, valb, colf, ridxf, valf, tb, cbv, st):
        cid = lax.axis_index("c")
        sid = lax.axis_index("s")
        wid = 2 * sid + cid
        z16i = jnp.zeros((_L,), jnp.int32)
        z16f = jnp.zeros((_L,), jnp.float32)
        iota = lax.iota(jnp.int32, _L)

        for i in range(nch * _STR // _L):
            colf[pl.ds(i * _L, _L)] = z16i
            ridxf[pl.ds(i * _L, _L)] = z16i
            valf[pl.ds(i * _L, _L)] = z16f
        for k in range(nch):
            st[k] = 0
            st[16 + k] = 0

        def flush(k):
            for i in range(_CAP // _L):
                tb[0, pl.ds(i * _L, _L)] = colf[pl.ds(k * _STR + i * _L, _L)]
                tb[1, pl.ds(i * _L, _L)] = ridxf[pl.ds(k * _STR + i * _L, _L)]
                tb[2, pl.ds(i * _L, _L)] = plsc.bitcast(
                    valf[pl.ds(k * _STR + i * _L, _L)], jnp.int32)
            nb = st[16 + k]
            pltpu.sync_copy(tb, bins_h.at[wid, k, nb])
            st[16 + k] = nb + 1
            for i in range(_CAP // _L):
                valf[pl.ds(k * _STR + i * _L, _L)] = z16f
            st[k] = 0

        off0 = wid * span

        def _grp(g, c):
            rv = rowb[pl.ds(g * _L, _L)]
            cv = colb[pl.ds(g * _L, _L)]
            vv = valb[pl.ds(g * _L, _L)]
            for k in range(nch):
                base = k * chunk
                m = (rv >= base) & (rv < base + chunk)

                @pl.when(st[k] > _CAP - _L)
                def _():
                    flush(k)

                cnt = st[k]
                cs = lax.cumsum(m.astype(jnp.int32))
                pos = jnp.where(m, k * _STR + cnt + cs - 1, k * _STR + _CAP)
                plsc.store_scatter(colf, [pos], cv)
                plsc.store_scatter(ridxf, [pos], rv - base)
                plsc.store_scatter(valf, [pos], vv)
                st[k] = cnt + cs[_L - 1]
            return c

        def _blk(b, c):
            off = off0 + b * _BLK
            pltpu.sync_copy(row_h.at[pl.ds(off, _BLK)], rowb)
            pltpu.sync_copy(col_h.at[pl.ds(off, _BLK)], colb)
            pltpu.sync_copy(val_h.at[pl.ds(off, _BLK)], valb)
            lax.fori_loop(0, ngrp, _grp, 0)
            return c
        lax.fori_loop(0, nblk, _blk, 0)

        for k in range(nch):
            @pl.when(st[k] > 0)
            def _():
                flush(k)

        for i in range(_CAP // _L):
            tb[0, pl.ds(i * _L, _L)] = z16i
            tb[1, pl.ds(i * _L, _L)] = z16i
            tb[2, pl.ds(i * _L, _L)] = z16i
        for k in range(nch):
            for _extra in range(3):
                @pl.when(lax.rem(st[16 + k], 4) != 0)
                def _():
                    pltpu.sync_copy(tb, bins_h.at[wid, k, st[16 + k]])
                    st[16 + k] = st[16 + k] + 1

        cvec = z16i
        for k in range(nch):
            cvec = jnp.where(iota == k, st[16 + k], cvec)
        cbv[pl.ds(0, _L)] = cvec
        pltpu.sync_copy(cbv, counts_h.at[wid])

    return binner


def _make_layer(nch, chunk, cap_b, n_out_pad):
    cps = nch // _NC
    rows_per_tile = chunk // _NS
    zfull, zrem = divmod(rows_per_tile, _CAP)
    mesh = plsc.VectorSubcoreMesh(core_axis_name="c", subcore_axis_name="s")

    @functools.partial(
        pl.kernel,
        out_type=jax.ShapeDtypeStruct((n_out_pad, _D), jnp.float32),
        mesh=mesh,
        compiler_params=_CP,
        scratch_types=[
            pltpu.VMEM_SHARED((chunk, _D), jnp.float32),
            pltpu.VMEM((4, 8, _CAP), jnp.int32),
            pltpu.VMEM((4 * _CAP, _D), jnp.float32),
            pltpu.VMEM((16,), jnp.int32),
            pltpu.SemaphoreType.DMA,
            pltpu.SemaphoreType.DMA,
            pltpu.SemaphoreType.DMA,
        ],
    )
    def layer(bins_h, counts_h, x_h, out_h, accum, tbs, G, cbv,
              sem_t, sem_g, sem_s):
        cid = lax.axis_index("c")
        sid = lax.axis_index("s")
        z16f = jnp.zeros((_L,), jnp.float32)
        iota = lax.iota(jnp.int32, _L)
        row0 = sid * rows_per_tile

        for k_i in range(cps):
            k = cid * cps + k_i
            def _zg(i, c):
                for d in range(_D // _L):
                    G[i, pl.ds(d * _L, _L)] = z16f
                return c
            lax.fori_loop(0, _CAP, _zg, 0)
            for z in range(zfull):
                pltpu.sync_copy(G.at[pl.ds(0, _CAP)],
                                accum.at[pl.ds(row0 + z * _CAP, _CAP)])
            if zrem:
                pltpu.sync_copy(G.at[pl.ds(0, zrem)],
                                accum.at[pl.ds(row0 + zfull * _CAP, zrem)])
            plsc.subcore_barrier()

            for j in range(2):
                w = 2 * sid + j
                pltpu.sync_copy(counts_h.at[w], cbv)
                nbv = cbv[pl.ds(0, _L)]
                nb = jnp.sum(jnp.where(iota == k, nbv, 0))
                ng = nb // 4

                def _grp4(g, c):
                    pltpu.sync_copy(bins_h.at[w, k, pl.ds(4 * g, 4)], tbs)

                    gd = [pltpu.async_copy(x_h.at[tbs.at[q, 0]],
                                           G.at[pl.ds(q * _CAP, _CAP)], sem_g)
                          for q in range(4)]
                    for d_ in gd:
                        d_.wait()

                    for q in range(4):
                        def _sc(g2, c2, q=q):
                            vv = plsc.bitcast(
                                tbs[q, 2, pl.ds(g2 * _L, _L)], jnp.float32)
                            for r in range(_L):
                                jrow = q * _CAP + g2 * _L + r
                                vs = z16f + vv[r]
                                for d in range(_D // _L):
                                    G[jrow, pl.ds(d * _L, _L)] = (
                                        G[jrow, pl.ds(d * _L, _L)] * vs)
                            return c2
                        lax.fori_loop(0, _CAP // _L, _sc, 0)

                    sd = [pltpu.async_copy(G.at[pl.ds(q * _CAP, _CAP)],
                                           accum.at[tbs.at[q, 1]],
                                           sem_s, add=True)
                          for q in range(4)]
                    for d_ in sd:
                        d_.wait()
                    return c
                lax.fori_loop(0, ng, _grp4, 0)
            plsc.subcore_barrier()

            base = k * chunk
            for z in range(zfull):
                pltpu.sync_copy(accum.at[pl.ds(row0 + z * _CAP, _CAP)],
                                out_h.at[pl.ds(base + row0 + z * _CAP, _CAP)])
            if zrem:
                pltpu.sync_copy(
                    accum.at[pl.ds(row0 + zfull * _CAP, zrem)],
                    out_h.at[pl.ds(base + row0 + zfull * _CAP, zrem)])

    return layer


_CAPB_A = (_NNZ_A_PAD // _NW) // 113 + 6
_CAPB_U = (_NNZ_U // _NW) // 113 + 6
_bin_adj = _make_bin(_NNZ_A_PAD, 8, 6400, _CAPB_A)
_bin_usr = _make_bin(_NNZ_U, 2, 5120, _CAPB_U)
_layer_adj = _make_layer(8, 6400, _CAPB_A, _NI_PAD)
_layer_usr = _make_layer(2, 5120, _CAPB_U, _NU_PAD)

_gmesh = plsc.VectorSubcoreMesh(core_axis_name="c", subcore_axis_name="s")


@functools.partial(
    pl.kernel,
    out_type=jax.ShapeDtypeStruct((1024, _D), jnp.float32),
    mesh=_gmesh,
    compiler_params=_CP,
    scratch_types=[
        pltpu.VMEM((32,), jnp.int32),
        pltpu.VMEM((32, _D), jnp.float32),
        pltpu.SemaphoreType.DMA,
    ],
)
def _gather_users(user_h, tab_h, out_h, idxb, g32, sem):
    wid = lax.axis_index("s") * _NC + lax.axis_index("c")
    b0 = wid * 32
    pltpu.sync_copy(user_h.at[pl.ds(b0, 32)], idxb)
    pltpu.async_copy(tab_h.at[idxb], g32, sem).wait()
    pltpu.sync_copy(g32, out_h.at[pl.ds(b0, 32)])


def _sum4(a, b, c, d):
    def body(a_r, b_r, c_r, d_r, o_r):
        o_r[...] = a_r[...] + b_r[...] + c_r[...] + d_r[...]
    n = a.shape[0]
    blkr = 512
    return pl.pallas_call(
        body,
        grid=(n // blkr,),
        in_specs=[pl.BlockSpec((blkr, _D), lambda i: (i, 0))] * 4,
        out_specs=pl.BlockSpec((blkr, _D), lambda i: (i, 0)),
        out_shape=jax.ShapeDtypeStruct((n, _D), jnp.float32),
    )(a, b, c, d)


def kernel(adj_row, adj_col, adj_val, u_row, u_col, u_val, ishist,
           hist_item, hist_len, embedding, user_embedding, user):
    adj_row = adj_row.astype(jnp.int32)
    adj_col = adj_col.astype(jnp.int32)
    u_row = u_row.astype(jnp.int32)
    u_col = u_col.astype(jnp.int32)
    user = user.astype(jnp.int32)

    npad = _NNZ_A_PAD - _NNZ_A
    adj_row_p = jnp.pad(adj_row, (0, npad), constant_values=_NI_PAD)
    adj_col_p = jnp.pad(adj_col, (0, npad))
    adj_val_p = jnp.pad(adj_val, (0, npad))

    x0 = jnp.pad(embedding, ((0, _NI_PAD - _N_ITEMS), (0, _D - _EMB)))

    bins_a, counts_a = _bin_adj(adj_row_p, adj_col_p, adj_val_p)
    c1 = _layer_adj(bins_a, counts_a, x0)
    c2 = _layer_adj(bins_a, counts_a, c1)
    c3 = _layer_adj(bins_a, counts_a, c2)
    fin = _sum4(x0, c1, c2, c3)

    bins_u, counts_u = _bin_usr(u_row, u_col, u_val)
    utab = _layer_usr(bins_u, counts_u, fin)
    ue = _gather_users(user, utab)
    return fin[:_N_ITEMS, :_EMB], ue[:, :_EMB]


# speedup vs baseline: 2.3722x; 2.3711x over previous
import functools

import jax
import jax.numpy as jnp
from jax import lax
from jax.experimental import pallas as pl
from jax.experimental.pallas import tpu as pltpu
from jax.experimental.pallas import tpu_sc as plsc

_N_ITEMS = 50000
_N_USERS = 10000
_EMB = 100
_D = 128
_NI_PAD = 51200
_NU_PAD = 10240
_NC = 2
_NS = 16
_L = 16
_BLK = 2000
_CAP = 128
_CP = pltpu.CompilerParams(needs_layout_passes=False)


def _make_spmm(nnz, n_out_pad, chunk, chunks_per_sc):
    span = nnz // _NS
    nblk = span // _BLK
    ngrp = _BLK // _L
    rows_per_tile = chunk // _NS
    assert span % _BLK == 0 and _BLK % _L == 0 and rows_per_tile % 16 == 0
    mesh = plsc.VectorSubcoreMesh(core_axis_name="c", subcore_axis_name="s")

    @functools.partial(
        pl.kernel,
        out_type=jax.ShapeDtypeStruct((n_out_pad, _D), jnp.float32),
        mesh=mesh,
        compiler_params=_CP,
        scratch_types=[
            pltpu.VMEM_SHARED((chunk, _D), jnp.float32),
            pltpu.VMEM((_BLK,), jnp.int32),
            pltpu.VMEM((_BLK,), jnp.int32),
            pltpu.VMEM((_BLK,), jnp.float32),
            pltpu.VMEM((_CAP + _L,), jnp.int32),
            pltpu.VMEM((_CAP + _L,), jnp.int32),
            pltpu.VMEM((_CAP + _L,), jnp.float32),
            pltpu.VMEM((_CAP,), jnp.int32),
            pltpu.VMEM((_CAP,), jnp.int32),
            pltpu.VMEM((_CAP,), jnp.float32),
            pltpu.VMEM((_CAP, _D), jnp.float32),
            pltpu.SMEM((8,), jnp.int32),
            pltpu.SemaphoreType.DMA,
        ],
    )
    def spmm(row_h, col_h, val_h, x_h, out_h,
             accum, rowb, colb, valb, colf, ridxf, valf, colc, ridxc, valc,
             G, cnt_ref, sem):
        cid = lax.axis_index("c")
        sid = lax.axis_index("s")
        z16i = jnp.zeros((_L,), jnp.int32)
        z16f = jnp.zeros((_L,), jnp.float32)

        for i in range((_CAP + _L) // _L):
            colf[pl.ds(i * _L, _L)] = z16i
            ridxf[pl.ds(i * _L, _L)] = z16i
            valf[pl.ds(i * _L, _L)] = z16f
        cnt_ref[0] = 0
        cnt_ref[1] = 0

        def issue():
            for i in range(_CAP // _L):
                colc[pl.ds(i * _L, _L)] = colf[pl.ds(i * _L, _L)]
                ridxc[pl.ds(i * _L, _L)] = ridxf[pl.ds(i * _L, _L)]
                valc[pl.ds(i * _L, _L)] = valf[pl.ds(i * _L, _L)]
            pltpu.async_copy(x_h.at[colc], G, sem)
            for i in range(_CAP // _L):
                valf[pl.ds(i * _L, _L)] = z16f
            cnt_ref[0] = 0
            cnt_ref[1] = 1

        def complete():
            pltpu.make_async_copy(x_h.at[colc], G, sem).wait()

            def _scale(g, c):
                vv = valc[pl.ds(g * _L, _L)]
                for r in range(_L):
                    j = g * _L + r
                    vs = z16f + vv[r]
                    for d in range(_D // _L):
                        G[j, pl.ds(d * _L, _L)] = G[j, pl.ds(d * _L, _L)] * vs
                return c
            lax.fori_loop(0, _CAP // _L, _scale, 0)

            pltpu.sync_copy(G, accum.at[ridxc], add=True)
            cnt_ref[1] = 0

        zfull, zrem = divmod(rows_per_tile, _CAP)
        for ci in range(chunks_per_sc):
            base = (cid * chunks_per_sc + ci) * chunk
            row0 = sid * rows_per_tile
            def _zg(i, c):
                for d in range(_D // _L):
                    G[i, pl.ds(d * _L, _L)] = z16f
                return c
            lax.fori_loop(0, _CAP, _zg, 0)
            for z in range(zfull):
                pltpu.sync_copy(G, accum.at[pl.ds(row0 + z * _CAP, _CAP)])
            if zrem:
                pltpu.sync_copy(G.at[pl.ds(0, zrem)],
                                accum.at[pl.ds(row0 + zfull * _CAP, zrem)])
            plsc.subcore_barrier()

            tile_lo = sid * span

            def _grp(g, c):
                rv = rowb[pl.ds(g * _L, _L)]
                m = (rv >= base) & (rv < base + chunk)

                @pl.when(cnt_ref[0] > _CAP - _L)
                def _():
                    @pl.when(cnt_ref[1] == 1)
                    def _():
                        complete()
                    issue()

                cnt = cnt_ref[0]
                cs = lax.cumsum(m.astype(jnp.int32))
                pos = jnp.where(m, cnt + cs - 1, _CAP)
                plsc.store_scatter(colf, [pos], colb[pl.ds(g * _L, _L)])
                plsc.store_scatter(ridxf, [pos], rv - base)
                plsc.store_scatter(valf, [pos], valb[pl.ds(g * _L, _L)])
                cnt_ref[0] = cnt + cs[_L - 1]
                return c

            def _blk(b, c):
                off = tile_lo + b * _BLK
                pltpu.sync_copy(row_h.at[pl.ds(off, _BLK)], rowb)
                pltpu.sync_copy(col_h.at[pl.ds(off, _BLK)], colb)
                pltpu.sync_copy(val_h.at[pl.ds(off, _BLK)], valb)
                lax.fori_loop(0, ngrp, _grp, 0)
                return c
            lax.fori_loop(0, nblk, _blk, 0)

            @pl.when(cnt_ref[1] == 1)
            def _():
                complete()

            @pl.when(cnt_ref[0] > 0)
            def _():
                issue()
                complete()
            plsc.subcore_barrier()

            for z in range(zfull):
                pltpu.sync_copy(accum.at[pl.ds(row0 + z * _CAP, _CAP)],
                                out_h.at[pl.ds(base + row0 + z * _CAP, _CAP)])
            if zrem:
                pltpu.sync_copy(
                    accum.at[pl.ds(row0 + zfull * _CAP, zrem)],
                    out_h.at[pl.ds(base + row0 + zfull * _CAP, zrem)])

    return spmm


_spmm_adj = _make_spmm(800000, _NI_PAD, 12800, 2)
_spmm_usr = _make_spmm(320000, _NU_PAD, 5120, 1)

_gmesh = plsc.VectorSubcoreMesh(core_axis_name="c", subcore_axis_name="s")


@functools.partial(
    pl.kernel,
    out_type=jax.ShapeDtypeStruct((1024, _D), jnp.float32),
    mesh=_gmesh,
    compiler_params=_CP,
    scratch_types=[
        pltpu.VMEM((32,), jnp.int32),
        pltpu.VMEM((32, _D), jnp.float32),
        pltpu.SemaphoreType.DMA,
    ],
)
def _gather_users(user_h, tab_h, out_h, idxb, g32, sem):
    wid = lax.axis_index("s") * _NC + lax.axis_index("c")
    b0 = wid * 32
    pltpu.sync_copy(user_h.at[pl.ds(b0, 32)], idxb)
    pltpu.async_copy(tab_h.at[idxb], g32, sem).wait()
    pltpu.sync_copy(g32, out_h.at[pl.ds(b0, 32)])


def _sum4(a, b, c, d):
    def body(a_r, b_r, c_r, d_r, o_r):
        o_r[...] = a_r[...] + b_r[...] + c_r[...] + d_r[...]
    n = a.shape[0]
    blkr = 512
    return pl.pallas_call(
        body,
        grid=(n // blkr,),
        in_specs=[pl.BlockSpec((blkr, 128), lambda i: (i, 0))] * 4,
        out_specs=pl.BlockSpec((blkr, 128), lambda i: (i, 0)),
        out_shape=jax.ShapeDtypeStruct((n, 128), jnp.float32),
    )(a, b, c, d)


def kernel(adj_row, adj_col, adj_val, u_row, u_col, u_val, ishist,
           hist_item, hist_len, embedding, user_embedding, user):
    adj_row = adj_row.astype(jnp.int32)
    adj_col = adj_col.astype(jnp.int32)
    u_row = u_row.astype(jnp.int32)
    u_col = u_col.astype(jnp.int32)
    user = user.astype(jnp.int32)

    x0 = jnp.pad(embedding, ((0, _NI_PAD - _N_ITEMS), (0, _D - _EMB)))
    c1 = _spmm_adj(adj_row, adj_col, adj_val, x0)
    c2 = _spmm_adj(adj_row, adj_col, adj_val, c1)
    c3 = _spmm_adj(adj_row, adj_col, adj_val, c2)
    fin = _sum4(x0, c1, c2, c3)
    utab = _spmm_usr(u_row, u_col, u_val, fin)
    ue = _gather_users(user, utab)
    return fin[:_N_ITEMS, :_EMB], ue[:, :_EMB]
